# baseline (speedup 1.0000x reference)
import functools

import jax
import jax.numpy as jnp
from jax import lax
from jax.experimental import pallas as pl
from jax.experimental.pallas import tpu as pltpu
from jax.experimental.pallas import tpu_sc as plsc

_NUM_CHANNELS = 1000000
_D = 32
_BATCH = 16384
_HIST = 200
_N = _BATCH * _HIST
_IW = 128
_NROWS = _N // _IW
_NC = 2
_NS = 16
_NW = _NC * _NS
_RPW = _NROWS // _NW
_G = 4
_PAIRS = _RPW // (2 * _G)
_BPH = _BATCH // _IW


def _sc_fused(ids2d, table):
    mesh = plsc.VectorSubcoreMesh(
        core_axis_name="c", subcore_axis_name="s",
        num_cores=_NC, num_subcores=_NS)

    @functools.partial(
        pl.kernel,
        out_type=jax.ShapeDtypeStruct((_HIST, _D // 8, _BPH, 8, _IW),
                                      jnp.float32),
        mesh=mesh,
        scratch_types=[
            pltpu.VMEM((_G, _IW), jnp.int32),
            pltpu.VMEM((_G, _IW), jnp.int32),
            pltpu.VMEM((_G * _IW, _D), jnp.float32),
            pltpu.VMEM((_G * _IW, _D), jnp.float32),
            pltpu.VMEM((_G, _D // 8, 8, _IW), jnp.float32),
            pltpu.SemaphoreType.DMA,
            pltpu.SemaphoreType.DMA,
        ],
        compiler_params=pltpu.CompilerParams(
            use_tc_tiling_on_sc=False, needs_layout_passes=False),
    )
    def k(idx_hbm, table_hbm, out_hbm, idx_a, idx_b, gb_a, gb_b, sbuf,
          sem_a, sem_b):
        wid = lax.axis_index("s") * _NC + lax.axis_index("c")
        row0 = wid * _RPW
        lanes = jax.lax.iota(jnp.int32, 16)

        def load_clamp_fire(base, idx_v, gb, sem):
            pltpu.sync_copy(idx_hbm.at[pl.ds(base, _G)], idx_v)

            def _clamp_row(j, _):
                def _clamp16(t, _):
                    v = idx_v[j, pl.ds(t * 16, 16)]
                    v = jnp.minimum(jnp.maximum(v, 0), _NUM_CHANNELS - 1)
                    idx_v[j, pl.ds(t * 16, 16)] = v
                    return 0
                return lax.fori_loop(0, _IW // 16, _clamp16, 0)

            lax.fori_loop(0, _G, _clamp_row, 0)
            for g in range(_G):
                pltpu.async_copy(
                    table_hbm.at[idx_v.at[g]],
                    gb.at[pl.ds(g * _IW, _IW)],
                    sem,
                )

        def drain(idx_v, gb, sem):
            for g in range(_G):
                pltpu.make_async_copy(
                    table_hbm.at[idx_v.at[g]],
                    gb.at[pl.ds(g * _IW, _IW)],
                    sem,
                ).wait()

        def process(base, gb):
            hblk = base // (_BPH * 8)
            bblk = (base // 8) % _BPH
            hsub0 = base % 8
            for g in range(_G):
                rows = [lanes + (g * _IW + bg * 16) for bg in range(8)]

                @plsc.parallel_loop(0, _D, unroll=8)
                def _trans_d(d):
                    dv = jnp.full((16,), d, jnp.int32)
                    for bg in range(8):
                        v = plsc.load_gather(gb, [rows[bg], dv])
                        sbuf[g, d // 8, d % 8, pl.ds(bg * 16, 16)] = v
            for g in range(_G):
                h = hblk * 8 + hsub0 + g
                pltpu.sync_copy(sbuf.at[g], out_hbm.at[h, :, bblk])

        load_clamp_fire(row0, idx_a, gb_a, sem_a)

        @pl.loop(0, _PAIRS)
        def _pair(t):
            base_e = row0 + (2 * t) * _G
            load_clamp_fire(base_e + _G, idx_b, gb_b, sem_b)
            drain(idx_a, gb_a, sem_a)
            process(base_e, gb_a)

            @pl.when(t < _PAIRS - 1)
            def _():
                load_clamp_fire(base_e + 2 * _G, idx_a, gb_a, sem_a)

            drain(idx_b, gb_b, sem_b)
            process(base_e + _G, gb_b)

    return k(ids2d, table)


def kernel(channel_ids, table):
    ids2d = (channel_ids.T
             .reshape(_HIST // 8, 8, _BPH, _IW)
             .transpose(0, 2, 1, 3)
             .reshape(_NROWS, _IW))
    out5 = _sc_fused(ids2d, table)
    return out5.transpose(2, 4, 0, 1, 3).reshape(_BATCH, _HIST, _D)


# speedup vs baseline: 1.0649x; 1.0649x over previous
import functools

import jax
import jax.numpy as jnp
from jax import lax
from jax.experimental import pallas as pl
from jax.experimental.pallas import tpu as pltpu
from jax.experimental.pallas import tpu_sc as plsc

_NUM_CHANNELS = 1000000
_D = 32
_BATCH = 16384
_HIST = 200
_N = _BATCH * _HIST
_IW = 128
_NROWS = _N // _IW
_NC = 2
_NS = 16
_NW = _NC * _NS
_RPW = _NROWS // _NW
_G = 8
_PAIRS = _RPW // (2 * _G)
_BPH = _BATCH // _IW


def _sc_fused(ids2d, table):
    mesh = plsc.VectorSubcoreMesh(
        core_axis_name="c", subcore_axis_name="s",
        num_cores=_NC, num_subcores=_NS)

    @functools.partial(
        pl.kernel,
        out_type=jax.ShapeDtypeStruct((_HIST, _D // 8, _BPH, 8, _IW),
                                      jnp.float32),
        mesh=mesh,
        scratch_types=[
            pltpu.VMEM((_G, _IW), jnp.int32),
            pltpu.VMEM((_G, _IW), jnp.int32),
            pltpu.VMEM((_G * _IW, _D), jnp.float32),
            pltpu.VMEM((_G * _IW, _D), jnp.float32),
            pltpu.VMEM((_G, _D // 8, 8, _IW), jnp.float32),
            pltpu.SemaphoreType.DMA,
            pltpu.SemaphoreType.DMA,
        ],
        compiler_params=pltpu.CompilerParams(
            use_tc_tiling_on_sc=False, needs_layout_passes=False),
    )
    def k(idx_hbm, table_hbm, out_hbm, idx_a, idx_b, gb_a, gb_b, sbuf,
          sem_a, sem_b):
        wid = lax.axis_index("s") * _NC + lax.axis_index("c")
        row0 = wid * _RPW
        lanes = jax.lax.iota(jnp.int32, 16)

        def load_clamp_fire(base, idx_v, gb, sem):
            pltpu.sync_copy(idx_hbm.at[pl.ds(base, _G)], idx_v)

            def _clamp_row(j, _):
                def _clamp16(t, _):
                    v = idx_v[j, pl.ds(t * 16, 16)]
                    v = jnp.minimum(jnp.maximum(v, 0), _NUM_CHANNELS - 1)
                    idx_v[j, pl.ds(t * 16, 16)] = v
                    return 0
                return lax.fori_loop(0, _IW // 16, _clamp16, 0)

            lax.fori_loop(0, _G, _clamp_row, 0)
            for g in range(_G):
                pltpu.async_copy(
                    table_hbm.at[idx_v.at[g]],
                    gb.at[pl.ds(g * _IW, _IW)],
                    sem,
                )

        def drain(idx_v, gb, sem):
            for g in range(_G):
                pltpu.make_async_copy(
                    table_hbm.at[idx_v.at[g]],
                    gb.at[pl.ds(g * _IW, _IW)],
                    sem,
                ).wait()

        def process(base, gb):
            hblk = base // (_BPH * 8)
            bblk = (base // 8) % _BPH
            hsub0 = base % 8
            for g in range(_G):
                rows = [lanes + (g * _IW + bg * 16) for bg in range(8)]

                @plsc.parallel_loop(0, _D, unroll=4)
                def _trans_d(d):
                    dv = jnp.full((16,), d, jnp.int32)
                    for bg in range(8):
                        v = plsc.load_gather(gb, [rows[bg], dv])
                        sbuf[g, d // 8, d % 8, pl.ds(bg * 16, 16)] = v
            for g in range(_G):
                h = hblk * 8 + hsub0 + g
                pltpu.sync_copy(sbuf.at[g], out_hbm.at[h, :, bblk])

        load_clamp_fire(row0, idx_a, gb_a, sem_a)

        @pl.loop(0, _PAIRS)
        def _pair(t):
            base_e = row0 + (2 * t) * _G
            load_clamp_fire(base_e + _G, idx_b, gb_b, sem_b)
            drain(idx_a, gb_a, sem_a)
            process(base_e, gb_a)

            @pl.when(t < _PAIRS - 1)
            def _():
                load_clamp_fire(base_e + 2 * _G, idx_a, gb_a, sem_a)

            drain(idx_b, gb_b, sem_b)
            process(base_e + _G, gb_b)

    return k(ids2d, table)


def kernel(channel_ids, table):
    ids2d = (channel_ids.T
             .reshape(_HIST // 8, 8, _BPH, _IW)
             .transpose(0, 2, 1, 3)
             .reshape(_NROWS, _IW))
    out5 = _sc_fused(ids2d, table)
    return out5.transpose(2, 4, 0, 1, 3).reshape(_BATCH, _HIST, _D)


# speedup vs baseline: 1.0727x; 1.0073x over previous
import functools

import jax
import jax.numpy as jnp
from jax import lax
from jax.experimental import pallas as pl
from jax.experimental.pallas import tpu as pltpu
from jax.experimental.pallas import tpu_sc as plsc

_NUM_CHANNELS = 1000000
_D = 32
_BATCH = 16384
_HIST = 200
_N = _BATCH * _HIST
_IW = 128
_NROWS = _N // _IW
_NC = 2
_NS = 16
_NW = _NC * _NS
_RPW = _NROWS // _NW
_G = 8
_PAIRS = _RPW // (2 * _G)
_BPH = _BATCH // _IW


def _sc_fused(ids2d, table):
    mesh = plsc.VectorSubcoreMesh(
        core_axis_name="c", subcore_axis_name="s",
        num_cores=_NC, num_subcores=_NS)

    @functools.partial(
        pl.kernel,
        out_type=jax.ShapeDtypeStruct((_HIST, _D // 8, _BPH, 8, _IW),
                                      jnp.float32),
        mesh=mesh,
        scratch_types=[
            pltpu.VMEM((_G, _IW), jnp.int32),
            pltpu.VMEM((_G, _IW), jnp.int32),
            pltpu.VMEM((_G * _IW, _D), jnp.float32),
            pltpu.VMEM((_G * _IW, _D), jnp.float32),
            pltpu.VMEM((_G, _D // 8, 8, _IW), jnp.float32),
            pltpu.SemaphoreType.DMA,
            pltpu.SemaphoreType.DMA,
        ],
        compiler_params=pltpu.CompilerParams(
            use_tc_tiling_on_sc=False, needs_layout_passes=False),
    )
    def k(idx_hbm, table_hbm, out_hbm, idx_a, idx_b, gb_a, gb_b, sbuf,
          sem_a, sem_b):
        wid = lax.axis_index("s") * _NC + lax.axis_index("c")
        row0 = wid * _RPW
        lanes = jax.lax.iota(jnp.int32, 16)

        def load_clamp_fire(base, idx_v, gb, sem):
            pltpu.sync_copy(idx_hbm.at[pl.ds(base, _G)], idx_v)

            def _clamp_row(j, _):
                def _clamp16(t, _):
                    v = idx_v[j, pl.ds(t * 16, 16)]
                    v = jnp.minimum(jnp.maximum(v, 0), _NUM_CHANNELS - 1)
                    idx_v[j, pl.ds(t * 16, 16)] = v
                    return 0
                return lax.fori_loop(0, _IW // 16, _clamp16, 0)

            lax.fori_loop(0, _G, _clamp_row, 0)
            for g in range(_G):
                pltpu.async_copy(
                    table_hbm.at[idx_v.at[g]],
                    gb.at[pl.ds(g * _IW, _IW)],
                    sem,
                )

        def drain(idx_v, gb, sem):
            for g in range(_G):
                pltpu.make_async_copy(
                    table_hbm.at[idx_v.at[g]],
                    gb.at[pl.ds(g * _IW, _IW)],
                    sem,
                ).wait()

        def process(base, gb):
            hblk = base // (_BPH * 8)
            bblk = (base // 8) % _BPH
            hsub0 = base % 8
            @plsc.parallel_loop(0, _G * _D, unroll=4)
            def _trans(i):
                g = i // _D
                d = i % _D
                dv = jnp.full((16,), d, jnp.int32)
                base = g * _IW
                for bg in range(8):
                    v = plsc.load_gather(gb, [lanes + (base + bg * 16), dv])
                    sbuf[g, d // 8, d % 8, pl.ds(bg * 16, 16)] = v
            for g in range(_G):
                h = hblk * 8 + hsub0 + g
                pltpu.sync_copy(sbuf.at[g], out_hbm.at[h, :, bblk])

        load_clamp_fire(row0, idx_a, gb_a, sem_a)

        @pl.loop(0, _PAIRS)
        def _pair(t):
            base_e = row0 + (2 * t) * _G
            load_clamp_fire(base_e + _G, idx_b, gb_b, sem_b)
            drain(idx_a, gb_a, sem_a)
            process(base_e, gb_a)

            @pl.when(t < _PAIRS - 1)
            def _():
                load_clamp_fire(base_e + 2 * _G, idx_a, gb_a, sem_a)

            drain(idx_b, gb_b, sem_b)
            process(base_e + _G, gb_b)

    return k(ids2d, table)


def kernel(channel_ids, table):
    ids2d = (channel_ids.T
             .reshape(_HIST // 8, 8, _BPH, _IW)
             .transpose(0, 2, 1, 3)
             .reshape(_NROWS, _IW))
    out5 = _sc_fused(ids2d, table)
    return out5.transpose(2, 4, 0, 1, 3).reshape(_BATCH, _HIST, _D)
